# baseline (speedup 1.0000x reference)
import functools

import jax
import jax.numpy as jnp
from jax import lax
from jax.experimental import pallas as pl
from jax.experimental.pallas import tpu as pltpu
from jax.experimental.pallas import tpu_sc as plsc

_VOCAB = 1000000
_EMBED_DIM = 64
_AUX_DIM = 128
_TARGET_DIM = 64
_BATCH = 16384
_HIST = 50

_NC = 2
_NS = 16
_NW = _NC * _NS
_TOTAL = _BATCH * _HIST
_PER_W = _TOTAL // _NW
_CHUNK = 640
_NCHUNK = _PER_W // _CHUNK

_B_BLK = 128
_ROWS_BLK = _B_BLK * _HIST


@functools.partial(
    pl.kernel,
    out_type=jax.ShapeDtypeStruct((_TOTAL, _EMBED_DIM), jnp.float32),
    mesh=plsc.VectorSubcoreMesh(core_axis_name="c", subcore_axis_name="s"),
    scratch_types=[
        pltpu.VMEM((_PER_W,), jnp.int32),
        pltpu.VMEM((_CHUNK, _EMBED_DIM), jnp.float32),
        pltpu.VMEM((_CHUNK, _EMBED_DIM), jnp.float32),
        pltpu.SemaphoreType.DMA,
        pltpu.SemaphoreType.DMA,
        pltpu.SemaphoreType.DMA,
        pltpu.SemaphoreType.DMA,
    ],
    compiler_params=pltpu.CompilerParams(use_tc_tiling_on_sc=False),
)
def _sc_gather(table_hbm, idx_hbm, out_hbm, idx_v, rows0_v, rows1_v,
               sem_g0, sem_g1, sem_w0, sem_w1):
    wid = lax.axis_index("s") * _NC + lax.axis_index("c")
    base = wid * _PER_W

    pltpu.sync_copy(idx_hbm.at[pl.ds(pl.multiple_of(base, _PER_W), _PER_W)],
                    idx_v)

    def out_slice(k):
        return out_hbm.at[pl.ds(pl.multiple_of(base + k * _CHUNK, _CHUNK),
                                _CHUNK)]

    def step(j, carry):
        k0 = 2 * j
        k1 = 2 * j + 1

        @pl.when(j > 0)
        def _():
            pltpu.make_async_copy(rows0_v, out_slice(k0 - 2), sem_w0).wait()

        pltpu.async_copy(table_hbm.at[idx_v.at[pl.ds(k0 * _CHUNK, _CHUNK)]],
                         rows0_v, sem_g0)

        @pl.when(j > 0)
        def _():
            pltpu.make_async_copy(rows1_v, out_slice(k1 - 2), sem_w1).wait()

        pltpu.async_copy(table_hbm.at[idx_v.at[pl.ds(k1 * _CHUNK, _CHUNK)]],
                         rows1_v, sem_g1)

        pltpu.make_async_copy(
            table_hbm.at[idx_v.at[pl.ds(k0 * _CHUNK, _CHUNK)]],
            rows0_v, sem_g0).wait()
        pltpu.async_copy(rows0_v, out_slice(k0), sem_w0)

        pltpu.make_async_copy(
            table_hbm.at[idx_v.at[pl.ds(k1 * _CHUNK, _CHUNK)]],
            rows1_v, sem_g1).wait()
        pltpu.async_copy(rows1_v, out_slice(k1), sem_w1)
        return carry

    lax.fori_loop(0, _NCHUNK // 2, step, 0)
    pltpu.make_async_copy(rows0_v, out_slice(_NCHUNK - 2), sem_w0).wait()
    pltpu.make_async_copy(rows1_v, out_slice(_NCHUNK - 1), sem_w1).wait()


def _tc_body(x_ref, w1_ref, b1_ref, w2_ref, b2_ref, o_ref):
    m = lax.dot_general(
        w2_ref[...], w1_ref[...],
        dimension_numbers=(((1,), (0,)), ((), ())),
        preferred_element_type=jnp.float32,
    )
    c = lax.dot_general(
        b1_ref[...], w2_ref[...],
        dimension_numbers=(((1,), (1,)), ((), ())),
        preferred_element_type=jnp.float32,
    ) + b2_ref[...]
    o_ref[...] = lax.dot_general(
        x_ref[...], m,
        dimension_numbers=(((1,), (1,)), ((), ())),
        preferred_element_type=jnp.float32,
    ) + c


_VBLK = 8000


def _tc_transform(table, w1, b1, w2, b2):
    return pl.pallas_call(
        _tc_body,
        grid=(_VOCAB // _VBLK,),
        in_specs=[
            pl.BlockSpec((_VBLK, _EMBED_DIM), lambda i: (i, 0)),
            pl.BlockSpec((_AUX_DIM, _EMBED_DIM), lambda i: (0, 0)),
            pl.BlockSpec((1, _AUX_DIM), lambda i: (0, 0)),
            pl.BlockSpec((_TARGET_DIM, _AUX_DIM), lambda i: (0, 0)),
            pl.BlockSpec((1, _TARGET_DIM), lambda i: (0, 0)),
        ],
        out_specs=pl.BlockSpec((_VBLK, _TARGET_DIM), lambda i: (i, 0)),
        out_shape=jax.ShapeDtypeStruct((_VOCAB, _TARGET_DIM), jnp.float32),
    )(table, w1, b1, w2, b2)


def kernel(indices, table, W1, b1, W2, b2):
    idx = indices.astype(jnp.int32).reshape(_TOTAL)
    ytab = _tc_transform(table, W1, b1.reshape(1, _AUX_DIM), W2,
                         b2.reshape(1, _TARGET_DIM))
    out = _sc_gather(ytab, idx)
    return out.reshape(_BATCH, _HIST, _TARGET_DIM)


# speedup vs baseline: 1.1824x; 1.1824x over previous
import functools

import jax
import jax.numpy as jnp
from jax import lax
from jax.experimental import pallas as pl
from jax.experimental.pallas import tpu as pltpu
from jax.experimental.pallas import tpu_sc as plsc

_VOCAB = 1000000
_EMBED_DIM = 64
_AUX_DIM = 128
_TARGET_DIM = 64
_BATCH = 16384
_HIST = 50

_NC = 2
_NS = 16
_NW = _NC * _NS
_TOTAL = _BATCH * _HIST
_PER_W = _TOTAL // _NW
_CHUNK = 640
_NCHUNK = _PER_W // _CHUNK

_B_BLK = 256
_ROWS_BLK = _B_BLK * _HIST


@functools.partial(
    pl.kernel,
    out_type=jax.ShapeDtypeStruct((_TOTAL, _EMBED_DIM), jnp.float32),
    mesh=plsc.VectorSubcoreMesh(core_axis_name="c", subcore_axis_name="s"),
    scratch_types=[
        pltpu.VMEM((_PER_W,), jnp.int32),
        pltpu.VMEM((_CHUNK, _EMBED_DIM), jnp.float32),
        pltpu.VMEM((_CHUNK, _EMBED_DIM), jnp.float32),
        pltpu.SemaphoreType.DMA,
        pltpu.SemaphoreType.DMA,
        pltpu.SemaphoreType.DMA,
        pltpu.SemaphoreType.DMA,
    ],
    compiler_params=pltpu.CompilerParams(use_tc_tiling_on_sc=False),
)
def _sc_gather(table_hbm, idx_hbm, out_hbm, idx_v, rows0_v, rows1_v,
               sem_g0, sem_g1, sem_w0, sem_w1):
    wid = lax.axis_index("s") * _NC + lax.axis_index("c")
    base = wid * _PER_W

    pltpu.sync_copy(idx_hbm.at[pl.ds(pl.multiple_of(base, _PER_W), _PER_W)],
                    idx_v)

    def out_slice(k):
        return out_hbm.at[pl.ds(pl.multiple_of(base + k * _CHUNK, _CHUNK),
                                _CHUNK)]

    def step(j, carry):
        k0 = 2 * j
        k1 = 2 * j + 1

        @pl.when(j > 0)
        def _():
            pltpu.make_async_copy(rows0_v, out_slice(k0 - 2), sem_w0).wait()

        pltpu.async_copy(table_hbm.at[idx_v.at[pl.ds(k0 * _CHUNK, _CHUNK)]],
                         rows0_v, sem_g0)

        @pl.when(j > 0)
        def _():
            pltpu.make_async_copy(rows1_v, out_slice(k1 - 2), sem_w1).wait()

        pltpu.async_copy(table_hbm.at[idx_v.at[pl.ds(k1 * _CHUNK, _CHUNK)]],
                         rows1_v, sem_g1)

        pltpu.make_async_copy(
            table_hbm.at[idx_v.at[pl.ds(k0 * _CHUNK, _CHUNK)]],
            rows0_v, sem_g0).wait()
        pltpu.async_copy(rows0_v, out_slice(k0), sem_w0)

        pltpu.make_async_copy(
            table_hbm.at[idx_v.at[pl.ds(k1 * _CHUNK, _CHUNK)]],
            rows1_v, sem_g1).wait()
        pltpu.async_copy(rows1_v, out_slice(k1), sem_w1)
        return carry

    lax.fori_loop(0, _NCHUNK // 2, step, 0)
    pltpu.make_async_copy(rows0_v, out_slice(_NCHUNK - 2), sem_w0).wait()
    pltpu.make_async_copy(rows1_v, out_slice(_NCHUNK - 1), sem_w1).wait()


def _tc_body(x_ref, w1_ref, b1_ref, w2_ref, b2_ref, o_ref):
    m = lax.dot_general(
        w2_ref[...], w1_ref[...],
        dimension_numbers=(((1,), (0,)), ((), ())),
        preferred_element_type=jnp.float32,
    )
    c = lax.dot_general(
        b1_ref[...], w2_ref[...],
        dimension_numbers=(((1,), (1,)), ((), ())),
        preferred_element_type=jnp.float32,
    ) + b2_ref[...]
    xf = x_ref[...]
    ye = lax.dot_general(
        xf[:, 0:_EMBED_DIM], m,
        dimension_numbers=(((1,), (1,)), ((), ())),
        preferred_element_type=jnp.float32,
    ) + c
    yo = lax.dot_general(
        xf[:, _EMBED_DIM:2 * _EMBED_DIM], m,
        dimension_numbers=(((1,), (1,)), ((), ())),
        preferred_element_type=jnp.float32,
    ) + c
    ys = jnp.concatenate([ye.reshape(_ROWS_BLK // 2, 1, _TARGET_DIM),
                          yo.reshape(_ROWS_BLK // 2, 1, _TARGET_DIM)],
                         axis=1)
    o_ref[...] = ys.reshape(_B_BLK, _HIST, _TARGET_DIM)


def _tc_project(x, w1, b1, w2, b2):
    return pl.pallas_call(
        _tc_body,
        grid=(_BATCH // _B_BLK,),
        in_specs=[
            pl.BlockSpec((_ROWS_BLK // 2, 2 * _EMBED_DIM), lambda i: (i, 0)),
            pl.BlockSpec((_AUX_DIM, _EMBED_DIM), lambda i: (0, 0)),
            pl.BlockSpec((1, _AUX_DIM), lambda i: (0, 0)),
            pl.BlockSpec((_TARGET_DIM, _AUX_DIM), lambda i: (0, 0)),
            pl.BlockSpec((1, _TARGET_DIM), lambda i: (0, 0)),
        ],
        out_specs=pl.BlockSpec((_B_BLK, _HIST, _TARGET_DIM),
                               lambda i: (i, 0, 0)),
        out_shape=jax.ShapeDtypeStruct((_BATCH, _HIST, _TARGET_DIM),
                                       jnp.float32),
    )(x, w1, b1, w2, b2)


def kernel(indices, table, W1, b1, W2, b2):
    idx = indices.astype(jnp.int32).reshape(_TOTAL)
    gathered = _sc_gather(table, idx)
    paired = gathered.reshape(_TOTAL // 2, 2 * _EMBED_DIM)
    return _tc_project(paired, W1, b1.reshape(1, _AUX_DIM), W2,
                       b2.reshape(1, _TARGET_DIM))
